# Initial kernel scaffold; baseline (speedup 1.0000x reference)
import jax
import jax.numpy as jnp
from jax.experimental import pallas as pl


def kernel(x, edge_index, y, Ws, bs, Rws, Rbs):
    raise NotImplementedError("write your pallas kernel here")


# speedup vs baseline: 41.9764x; 41.9764x over previous
import functools

import jax
import jax.numpy as jnp
from jax import lax
from jax.experimental import pallas as pl
from jax.experimental.pallas import tpu as pltpu
from jax.experimental.pallas import tpu_sc as plsc

_NC = 2
_NS = 16
_NW = _NC * _NS
_K = 128
_GW = 8
_BN = 1000


def _make_sc_scatter_sum(n, ch, c_width, gather):
    mesh = plsc.VectorSubcoreMesh(core_axis_name="c", subcore_axis_name="s")
    n_pad = -(-n // (_NS * _K)) * (_NS * _K)
    rows_per_tile = n_pad // _NS
    nfull = rows_per_tile // _K
    rem = rows_per_tile % _K
    nseg = c_width // 16
    last_rows = n - (_NS - 1) * rows_per_tile
    assert 0 < last_rows <= rows_per_tile and last_rows % 8 == 0

    scratch = [
        pltpu.VMEM_SHARED((n_pad, c_width), jnp.float32),
        pltpu.VMEM((_K, c_width), jnp.float32),
        pltpu.VMEM((1, _K), jnp.int32),
    ]
    if gather:
        scratch += [
            pltpu.VMEM((1, _K), jnp.int32),
            pltpu.VMEM((_K, c_width), jnp.float32),
            pltpu.SemaphoreType.DMA,
        ]

    def body(*refs):
        if gather:
            src_r, dst_r, table, out, acc, zbuf, didx, sidx, rows, sem = refs
        else:
            dst_r, out, acc, zbuf, didx = refs
        c = lax.axis_index("c")
        s = lax.axis_index("s")
        w = s * _NC + c
        base = pl.multiple_of(s * rows_per_tile, _K)

        def fill(val):
            def row(i, _):
                def seg(k, _):
                    zbuf[i, pl.ds(k * 16, 16)] = jnp.full((16,), val, jnp.float32)
                    return 0
                return lax.fori_loop(0, nseg, seg, 0)
            lax.fori_loop(0, _K, row, 0)

        fill(0.0)
        for k in range(nfull):
            pltpu.sync_copy(zbuf, acc.at[pl.ds(base + k * _K, _K)])
        if rem:
            pltpu.sync_copy(zbuf.at[pl.ds(0, rem)],
                            acc.at[pl.ds(base + nfull * _K, rem)])
        if not gather:
            fill(1.0)
        plsc.subcore_barrier()

        lo = (ch * w) // _NW
        hi = (ch * (w + 1)) // _NW

        def chunk(j, _):
            pltpu.sync_copy(dst_r.at[j], didx.at[0])
            if gather:
                pltpu.sync_copy(src_r.at[j], sidx.at[0])
                pltpu.async_copy(table.at[sidx.at[0]], rows, sem).wait()
                pltpu.sync_copy(rows, acc.at[didx.at[0]], add=True)
            else:
                pltpu.sync_copy(zbuf, acc.at[didx.at[0]], add=True)
            return 0

        lax.fori_loop(lo, hi, chunk, 0)
        plsc.subcore_barrier()

        if last_rows == rows_per_tile:
            pltpu.sync_copy(acc.at[pl.ds(base, rows_per_tile)],
                            out.at[c, pl.ds(base, rows_per_tile)])
        else:
            @pl.when(s < _NS - 1)
            def _():
                pltpu.sync_copy(acc.at[pl.ds(base, rows_per_tile)],
                                out.at[c, pl.ds(base, rows_per_tile)])

            @pl.when(s == _NS - 1)
            def _():
                pltpu.sync_copy(acc.at[pl.ds(base, last_rows)],
                                out.at[c, pl.ds(base, last_rows)])

    return pl.kernel(
        body,
        out_type=jax.ShapeDtypeStruct((_NC, n, c_width), jnp.float32),
        mesh=mesh,
        scratch_types=scratch,
    )


def _tc1_body(degp_ref, x_ref, xs_ref, dinv_ref):
    deg = 1.0 + degp_ref[0][:, 0:1] + degp_ref[1][:, 0:1]
    dinv = lax.rsqrt(deg)
    dinv_ref[...] = dinv
    xs_ref[...] = x_ref[...] * dinv


def _tc2_body(nb, sp_ref, xs_ref, dinv_ref, w_ref, b_ref, rwp_ref, ms_ref):
    dinv = dinv_ref[...]
    t = dinv * (sp_ref[0] + sp_ref[1] + xs_ref[...])
    for i in range(nb):
        h = jnp.maximum(jnp.dot(t, w_ref[i]) + b_ref[i], 0.0)
        p = jnp.dot(h, w_ref[i])
        m = jnp.dot(p, rwp_ref[i])
        ms_ref[:, _GW * i:_GW * (i + 1)] = dinv * m
    ms_ref[:, _GW * nb:] = jnp.zeros((ms_ref.shape[0], ms_ref.shape[1] - _GW * nb),
                                     jnp.float32)


def _tc3_body(dims, s2_ref, ms_ref, dinv_ref, b_ref, rwp_ref, rbp_ref, out_ref):
    u = dinv_ref[...] * (s2_ref[0] + s2_ref[1] + ms_ref[...])
    bn = u.shape[0]
    for i, d in enumerate(dims):
        cvec = jnp.dot(b_ref[i].reshape(1, -1), rwp_ref[i]) + rbp_ref[i]
        z = u[:, _GW * i:_GW * (i + 1)] + cvec
        mask = lax.broadcasted_iota(jnp.int32, (bn, _GW), 1) < d
        z = jnp.where(mask, z, -1e30)
        mx = jnp.max(z, axis=1, keepdims=True)
        e = jnp.exp(z - mx)
        out_ref[:, _GW * i:_GW * (i + 1)] = e / jnp.sum(e, axis=1, keepdims=True)
    pad = out_ref.shape[1] - _GW * len(dims)
    if pad:
        out_ref[:, _GW * len(dims):] = jnp.zeros((bn, pad), jnp.float32)


def kernel(x, edge_index, y, Ws, bs, Rws, Rbs):
    n, h = x.shape
    e = edge_index.shape[1]
    nb = len(Ws)
    dims = [int(rw.shape[1]) for rw in Rws]
    ch = e // _K
    grid = (n // _BN,)
    msc_pad = 128

    src_r = edge_index[0].reshape(ch, _K)
    dst_r = edge_index[1].reshape(ch, _K)
    wstack = jnp.stack(Ws)
    bstack = jnp.stack(bs)
    rwp = jnp.stack([jnp.pad(rw, ((0, 0), (0, _GW - rw.shape[1])))
                     for rw in Rws])
    rbp = jnp.stack([jnp.pad(rb, (0, _GW - rb.shape[0])) for rb in Rbs])

    degp = _make_sc_scatter_sum(n, ch, 16, gather=False)(dst_r)

    xs, dinv = pl.pallas_call(
        _tc1_body,
        grid=grid,
        in_specs=[
            pl.BlockSpec((_NC, _BN, 16), lambda i: (0, i, 0)),
            pl.BlockSpec((_BN, h), lambda i: (i, 0)),
        ],
        out_specs=[
            pl.BlockSpec((_BN, h), lambda i: (i, 0)),
            pl.BlockSpec((_BN, 1), lambda i: (i, 0)),
        ],
        out_shape=[
            jax.ShapeDtypeStruct((n, h), jnp.float32),
            jax.ShapeDtypeStruct((n, 1), jnp.float32),
        ],
    )(degp, x)

    s1p = _make_sc_scatter_sum(n, ch, h, gather=True)(src_r, dst_r, xs)

    ms = pl.pallas_call(
        functools.partial(_tc2_body, nb),
        grid=grid,
        in_specs=[
            pl.BlockSpec((_NC, _BN, h), lambda i: (0, i, 0)),
            pl.BlockSpec((_BN, h), lambda i: (i, 0)),
            pl.BlockSpec((_BN, 1), lambda i: (i, 0)),
            pl.BlockSpec((nb, h, h), lambda i: (0, 0, 0)),
            pl.BlockSpec((nb, h), lambda i: (0, 0)),
            pl.BlockSpec((nb, h, _GW), lambda i: (0, 0, 0)),
        ],
        out_specs=pl.BlockSpec((_BN, msc_pad), lambda i: (i, 0)),
        out_shape=jax.ShapeDtypeStruct((n, msc_pad), jnp.float32),
    )(s1p, xs, dinv, wstack, bstack, rwp)

    s2p = _make_sc_scatter_sum(n, ch, msc_pad, gather=True)(src_r, dst_r, ms)

    out = pl.pallas_call(
        functools.partial(_tc3_body, dims),
        grid=grid,
        in_specs=[
            pl.BlockSpec((_NC, _BN, msc_pad), lambda i: (0, i, 0)),
            pl.BlockSpec((_BN, msc_pad), lambda i: (i, 0)),
            pl.BlockSpec((_BN, 1), lambda i: (i, 0)),
            pl.BlockSpec((nb, h), lambda i: (0, 0)),
            pl.BlockSpec((nb, h, _GW), lambda i: (0, 0, 0)),
            pl.BlockSpec((nb, _GW), lambda i: (0, 0)),
        ],
        out_specs=pl.BlockSpec((_BN, msc_pad), lambda i: (i, 0)),
        out_shape=jax.ShapeDtypeStruct((n, msc_pad), jnp.float32),
    )(s2p, ms, dinv, bstack, rwp, rbp)

    return tuple(out[:, _GW * i:_GW * i + d] for i, d in enumerate(dims))
